# baseline (speedup 1.0000x reference)
import functools

import jax
import jax.numpy as jnp
import numpy as np
from jax import lax
from jax.experimental import pallas as pl
from jax.experimental.pallas import tpu as pltpu
from jax.experimental.pallas import tpu_sc as plsc

_DIM = 33
_N3 = _DIM * _DIM * _DIM
_BINSIZE = np.float32(1.0001 / (_DIM - 1))
_INV_BIN = np.float32((_DIM - 1) / 1.0001)
_LANES = 16
_TR, _TC = 8, 128
_NW = 32
_UNROLL = 4


def _build_sc_call(B, H, W):
    nbatch = B
    wper = _NW // nbatch
    rows_per_w = H // wper
    nrt = rows_per_w // _TR
    nct = W // _TC
    nchunks = nrt * nct

    mesh = plsc.VectorSubcoreMesh(core_axis_name="c", subcore_axis_name="s")

    @functools.partial(
        pl.kernel,
        out_type=jax.ShapeDtypeStruct((B, 3, H, W), jnp.float32),
        mesh=mesh,
        compiler_params=pltpu.CompilerParams(needs_layout_passes=False),
        scratch_types=(
            [pltpu.VMEM((_N3,), jnp.float32)] * 3
            + [pltpu.VMEM((_TR, _TC), jnp.float32)] * 12
            + [pltpu.SemaphoreType.DMA] * 4
        ),
    )
    def sc_fn(l0_hbm, l1_hbm, l2_hbm, x_hbm, out_hbm, l0, l1, l2, *rest):
        lbufs = (l0, l1, l2)
        bufs = rest[:12]
        isems = rest[12:14]
        osems = rest[14:16]
        rbufs, gbufs, bbufs = bufs[0:2], bufs[2:4], bufs[4:6]
        obufs = [bufs[6:9], bufs[9:12]]

        cid = lax.axis_index("c")
        sid = lax.axis_index("s")
        wid = sid * 2 + cid
        batch = wid // wper
        quarter = wid % wper
        row0 = quarter * rows_per_w

        pltpu.sync_copy(l0_hbm, l0)
        pltpu.sync_copy(l1_hbm, l1)
        pltpu.sync_copy(l2_hbm, l2)

        def tile_at(ref, ch, chunk):
            rt = chunk >> 2
            ct = chunk & 3
            return ref.at[
                batch, ch, pl.ds(row0 + rt * _TR, _TR), pl.ds(ct * _TC, _TC)
            ]

        def start_in(chunk, slot):
            for c, buf in enumerate((rbufs[slot], gbufs[slot], bbufs[slot])):
                pltpu.async_copy(tile_at(x_hbm, c, chunk), buf, isems[slot])

        def wait_in(slot):
            for buf in (rbufs[slot], gbufs[slot], bbufs[slot]):
                pltpu.make_async_copy(tile_at(x_hbm, 0, 0), buf, isems[slot]).wait()

        def start_out(chunk, slot):
            for c in range(3):
                pltpu.async_copy(
                    obufs[slot][c], tile_at(out_hbm, c, chunk), osems[slot]
                )

        def wait_out(slot):
            for c in range(3):
                pltpu.make_async_copy(
                    obufs[slot][c], tile_at(out_hbm, 0, 0), osems[slot]
                ).wait()

        def compute(slot):
            rb, gb, bb = rbufs[slot], gbufs[slot], bbufs[slot]
            o0, o1, o2 = obufs[slot]

            @plsc.parallel_loop(0, _TR * _TC, step=_LANES, unroll=_UNROLL)
            def _group(s):
                row = s >> 7
                col = pl.multiple_of(s & (_TC - 1), _LANES)
                if True:
                    r = rb[row, pl.ds(col, _LANES)]
                    g = gb[row, pl.ds(col, _LANES)]
                    b = bb[row, pl.ds(col, _LANES)]
                    qr = r * _INV_BIN
                    qg = g * _INV_BIN
                    qb = b * _INV_BIN
                    rid = qr.astype(jnp.int32)
                    gid = qg.astype(jnp.int32)
                    bid = qb.astype(jnp.int32)
                    rd = qr - rid.astype(jnp.float32)
                    gd = qg - gid.astype(jnp.float32)
                    bd = qb - bid.astype(jnp.float32)
                    i000 = rid + gid * _DIM + bid * (_DIM * _DIM)

                    wr0 = 1.0 - rd
                    wg0 = 1.0 - gd
                    wb0 = 1.0 - bd
                    w00 = wg0 * wb0
                    w10 = gd * wb0
                    w01 = wg0 * bd
                    w11 = gd * bd
                    wc = (
                        (0, wr0 * w00),
                        (1, rd * w00),
                        (_DIM, wr0 * w10),
                        (_DIM + 1, rd * w10),
                        (_DIM * _DIM, wr0 * w01),
                        (_DIM * _DIM + 1, rd * w01),
                        (_DIM * _DIM + _DIM, wr0 * w11),
                        (_DIM * _DIM + _DIM + 1, rd * w11),
                    )
                    acc = [None, None, None]
                    for d, wk in wc:
                        j = i000 if d == 0 else i000 + d
                        for c in range(3):
                            v = plsc.load_gather(lbufs[c], [j])
                            t = wk * v
                            acc[c] = t if acc[c] is None else acc[c] + t

                    o0[row, pl.ds(col, _LANES)] = acc[0]
                    o1[row, pl.ds(col, _LANES)] = acc[1]
                    o2[row, pl.ds(col, _LANES)] = acc[2]

        start_in(0, 0)
        start_in(1, 1)

        @pl.loop(0, nchunks, step=2)
        def _pair(ci):
            for slot in range(2):
                chunk = ci + slot
                wait_in(slot)

                @pl.when(chunk >= 2)
                def _():
                    wait_out(slot)

                compute(slot)
                start_out(chunk, slot)

                @pl.when(chunk + 2 < nchunks)
                def _():
                    start_in(chunk + 2, slot)

        wait_out(0)
        wait_out(1)

    return sc_fn


def kernel(LUT, x):
    B, C, H, W = x.shape
    l0 = LUT[0].reshape(_N3)
    l1 = LUT[1].reshape(_N3)
    l2 = LUT[2].reshape(_N3)
    return _build_sc_call(B, H, W)(l0, l1, l2, x)


# speedup vs baseline: 1.3055x; 1.3055x over previous
import functools

import jax
import jax.numpy as jnp
import numpy as np
from jax import lax
from jax.experimental import pallas as pl
from jax.experimental.pallas import tpu as pltpu
from jax.experimental.pallas import tpu_sc as plsc

_DIM = 33
_N3 = _DIM * _DIM * _DIM
_BINSIZE = np.float32(1.0001 / (_DIM - 1))
_INV_BIN = np.float32((_DIM - 1) / 1.0001)
_LANES = 16
_TR, _TC = 8, 128
_NW = 32
_UNROLL = 3


def _build_sc_call(B, H, W):
    nbatch = B
    wper = _NW // nbatch
    rows_per_w = H // wper
    nrt = rows_per_w // _TR
    nct = W // _TC
    nchunks = nrt * nct

    mesh = plsc.VectorSubcoreMesh(core_axis_name="c", subcore_axis_name="s")

    @functools.partial(
        pl.kernel,
        out_type=jax.ShapeDtypeStruct((B, 3, H, W), jnp.float32),
        mesh=mesh,
        compiler_params=pltpu.CompilerParams(needs_layout_passes=False),
        scratch_types=(
            [pltpu.VMEM((_N3,), jnp.float32)] * 3
            + [pltpu.VMEM((_TR, _TC), jnp.float32)] * 12
            + [pltpu.SemaphoreType.DMA] * 4
        ),
    )
    def sc_fn(l0_hbm, l1_hbm, l2_hbm, x_hbm, out_hbm, l0, l1, l2, *rest):
        lbufs = (l0, l1, l2)
        bufs = rest[:12]
        isems = rest[12:14]
        osems = rest[14:16]
        rbufs, gbufs, bbufs = bufs[0:2], bufs[2:4], bufs[4:6]
        obufs = [bufs[6:9], bufs[9:12]]

        cid = lax.axis_index("c")
        sid = lax.axis_index("s")
        wid = sid * 2 + cid
        batch = wid // wper
        quarter = wid % wper
        row0 = quarter * rows_per_w

        pltpu.sync_copy(l0_hbm, l0)
        pltpu.sync_copy(l1_hbm, l1)
        pltpu.sync_copy(l2_hbm, l2)

        def tile_at(ref, ch, chunk):
            rt = chunk >> 2
            ct = chunk & 3
            return ref.at[
                batch, ch, pl.ds(row0 + rt * _TR, _TR), pl.ds(ct * _TC, _TC)
            ]

        def start_in(chunk, slot):
            for c, buf in enumerate((rbufs[slot], gbufs[slot], bbufs[slot])):
                pltpu.async_copy(tile_at(x_hbm, c, chunk), buf, isems[slot])

        def wait_in(slot):
            for buf in (rbufs[slot], gbufs[slot], bbufs[slot]):
                pltpu.make_async_copy(tile_at(x_hbm, 0, 0), buf, isems[slot]).wait()

        def start_out(chunk, slot):
            for c in range(3):
                pltpu.async_copy(
                    obufs[slot][c], tile_at(out_hbm, c, chunk), osems[slot]
                )

        def wait_out(slot):
            for c in range(3):
                pltpu.make_async_copy(
                    obufs[slot][c], tile_at(out_hbm, 0, 0), osems[slot]
                ).wait()

        def compute(slot):
            rb, gb, bb = rbufs[slot], gbufs[slot], bbufs[slot]
            o0, o1, o2 = obufs[slot]

            @plsc.parallel_loop(0, _TR * _TC, step=_LANES, unroll=_UNROLL)
            def _group(s):
                row = s >> 7
                col = pl.multiple_of(s & (_TC - 1), _LANES)
                if True:
                    r = rb[row, pl.ds(col, _LANES)]
                    g = gb[row, pl.ds(col, _LANES)]
                    b = bb[row, pl.ds(col, _LANES)]
                    qr = r * _INV_BIN
                    qg = g * _INV_BIN
                    qb = b * _INV_BIN
                    rid = qr.astype(jnp.int32)
                    gid = qg.astype(jnp.int32)
                    bid = qb.astype(jnp.int32)
                    rd = qr - rid.astype(jnp.float32)
                    gd = qg - gid.astype(jnp.float32)
                    bd = qb - bid.astype(jnp.float32)
                    i000 = rid + gid * _DIM + bid * (_DIM * _DIM)

                    wr0 = 1.0 - rd
                    wg0 = 1.0 - gd
                    wb0 = 1.0 - bd
                    w00 = wg0 * wb0
                    w10 = gd * wb0
                    w01 = wg0 * bd
                    w11 = gd * bd
                    wc = (
                        (0, wr0 * w00),
                        (1, rd * w00),
                        (_DIM, wr0 * w10),
                        (_DIM + 1, rd * w10),
                        (_DIM * _DIM, wr0 * w01),
                        (_DIM * _DIM + 1, rd * w01),
                        (_DIM * _DIM + _DIM, wr0 * w11),
                        (_DIM * _DIM + _DIM + 1, rd * w11),
                    )
                    acc = [None, None, None]
                    for d, wk in wc:
                        j = i000 if d == 0 else i000 + d
                        for c in range(3):
                            v = plsc.load_gather(lbufs[c], [j])
                            t = wk * v
                            acc[c] = t if acc[c] is None else acc[c] + t

                    o0[row, pl.ds(col, _LANES)] = acc[0]
                    o1[row, pl.ds(col, _LANES)] = acc[1]
                    o2[row, pl.ds(col, _LANES)] = acc[2]

        start_in(0, 0)
        start_in(1, 1)

        @pl.loop(0, nchunks, step=2)
        def _pair(ci):
            for slot in range(2):
                chunk = ci + slot
                wait_in(slot)

                @pl.when(chunk >= 2)
                def _():
                    wait_out(slot)

                compute(slot)
                start_out(chunk, slot)

                @pl.when(chunk + 2 < nchunks)
                def _():
                    start_in(chunk + 2, slot)

        wait_out(0)
        wait_out(1)

    return sc_fn


def kernel(LUT, x):
    B, C, H, W = x.shape
    l0 = LUT[0].reshape(_N3)
    l1 = LUT[1].reshape(_N3)
    l2 = LUT[2].reshape(_N3)
    return _build_sc_call(B, H, W)(l0, l1, l2, x)
